# baseline (speedup 1.0000x reference)
import jax
import jax.numpy as jnp
from jax.experimental import pallas as pl
from jax.experimental.pallas import tpu as pltpu

V_COARSE = 4096
V_FINE = 16384
COARSE_DIM = 256
FINE_INPUT_DIM = 256
OUTPUT_DIM = 256
SCALAR_PROJ_DIM = 64

ROWS = 512
NSPLIT = 8
CHUNK = ROWS // NSPLIT


def _fused_body(x_coarse_ref, w_sym_ref, b_sym_ref, *refs):
    interp_refs = refs[:NSPLIT]
    (x_fine_ref, w_fuse1_ref, w_fuse2_ref, b_fuse_ref,
     out_ref, x_scalar_ref) = refs[NSPLIT:]

    @pl.when(pl.program_id(0) == 0)
    def _():
        x_scalar_ref[...] = (
            jnp.dot(x_coarse_ref[...].astype(jnp.bfloat16),
                    w_sym_ref[...].astype(jnp.bfloat16),
                    preferred_element_type=jnp.float32)
            + b_sym_ref[...]
        ).astype(jnp.bfloat16)

    xs = x_scalar_ref[...]
    wf1 = w_fuse1_ref[...].astype(jnp.bfloat16)
    wf2 = w_fuse2_ref[...].astype(jnp.bfloat16)
    bf = b_fuse_ref[...]
    for j, iref in enumerate(interp_refs):
        t = jnp.dot(iref[...].astype(jnp.bfloat16), xs,
                    preferred_element_type=jnp.float32)
        rows = pl.ds(j * CHUNK, CHUNK)
        out_ref[rows, :] = (
            jnp.dot(t.astype(jnp.bfloat16), wf1,
                    preferred_element_type=jnp.float32)
            + jnp.dot(x_fine_ref[rows, :].astype(jnp.bfloat16), wf2,
                      preferred_element_type=jnp.float32)
            + bf
        )


def kernel(x_coarse, x_fine_input, interp_matrix, W_sym, b_sym, W_fuse, b_fuse):
    w_fuse1 = W_fuse[:SCALAR_PROJ_DIM, :]
    w_fuse2 = W_fuse[SCALAR_PROJ_DIM:, :]
    b_sym2 = b_sym.reshape(1, SCALAR_PROJ_DIM)
    b_fuse2 = b_fuse.reshape(1, OUTPUT_DIM)

    interp_specs = [
        pl.BlockSpec((CHUNK, V_COARSE), lambda i, j=j: (NSPLIT * i + j, 0))
        for j in range(NSPLIT)
    ]
    grid = (V_FINE // ROWS,)
    return pl.pallas_call(
        _fused_body,
        grid=grid,
        in_specs=[
            pl.BlockSpec((V_COARSE, COARSE_DIM), lambda i: (0, 0)),
            pl.BlockSpec((COARSE_DIM, SCALAR_PROJ_DIM), lambda i: (0, 0)),
            pl.BlockSpec((1, SCALAR_PROJ_DIM), lambda i: (0, 0)),
            *interp_specs,
            pl.BlockSpec((ROWS, FINE_INPUT_DIM), lambda i: (i, 0)),
            pl.BlockSpec((SCALAR_PROJ_DIM, OUTPUT_DIM), lambda i: (0, 0)),
            pl.BlockSpec((FINE_INPUT_DIM, OUTPUT_DIM), lambda i: (0, 0)),
            pl.BlockSpec((1, OUTPUT_DIM), lambda i: (0, 0)),
        ],
        out_specs=pl.BlockSpec((ROWS, OUTPUT_DIM), lambda i: (i, 0)),
        out_shape=jax.ShapeDtypeStruct((V_FINE, OUTPUT_DIM), jnp.float32),
        scratch_shapes=[pltpu.VMEM((V_COARSE, SCALAR_PROJ_DIM), jnp.bfloat16)],
        compiler_params=pltpu.CompilerParams(
            dimension_semantics=("arbitrary",)),
    )(x_coarse, W_sym, b_sym2, *([interp_matrix] * NSPLIT), x_fine_input,
      w_fuse1, w_fuse2, b_fuse2)


# speedup vs baseline: 1.0925x; 1.0925x over previous
import jax
import jax.numpy as jnp
from jax.experimental import pallas as pl
from jax.experimental.pallas import tpu as pltpu

V_COARSE = 4096
V_FINE = 16384
COARSE_DIM = 256
FINE_INPUT_DIM = 256
OUTPUT_DIM = 256
SCALAR_PROJ_DIM = 64

ROWS = 512


def _fused_body(x_coarse_ref, w_sym_ref, b_sym_ref, interp_ref, x_fine_ref,
                w_fuse1_ref, w_fuse2_ref, b_fuse_ref, out_ref, d_ref):
    @pl.when(pl.program_id(0) == 0)
    def _():
        xs = (
            jnp.dot(x_coarse_ref[...].astype(jnp.bfloat16),
                    w_sym_ref[...].astype(jnp.bfloat16),
                    preferred_element_type=jnp.float32)
            + b_sym_ref[...]
        )
        d_ref[...] = jnp.dot(xs.astype(jnp.bfloat16),
                             w_fuse1_ref[...].astype(jnp.bfloat16),
                             preferred_element_type=jnp.float32
                             ).astype(jnp.bfloat16)

    out_ref[...] = (
        jnp.dot(interp_ref[...].astype(jnp.bfloat16), d_ref[...],
                preferred_element_type=jnp.float32)
        + jnp.dot(x_fine_ref[...].astype(jnp.bfloat16),
                  w_fuse2_ref[...].astype(jnp.bfloat16),
                  preferred_element_type=jnp.float32)
        + b_fuse_ref[...]
    )


def kernel(x_coarse, x_fine_input, interp_matrix, W_sym, b_sym, W_fuse, b_fuse):
    w_fuse1 = W_fuse[:SCALAR_PROJ_DIM, :]
    w_fuse2 = W_fuse[SCALAR_PROJ_DIM:, :]
    b_sym2 = b_sym.reshape(1, SCALAR_PROJ_DIM)
    b_fuse2 = b_fuse.reshape(1, OUTPUT_DIM)

    grid = (V_FINE // ROWS,)
    return pl.pallas_call(
        _fused_body,
        grid=grid,
        in_specs=[
            pl.BlockSpec((V_COARSE, COARSE_DIM), lambda i: (0, 0)),
            pl.BlockSpec((COARSE_DIM, SCALAR_PROJ_DIM), lambda i: (0, 0)),
            pl.BlockSpec((1, SCALAR_PROJ_DIM), lambda i: (0, 0)),
            pl.BlockSpec((ROWS, V_COARSE), lambda i: (i, 0)),
            pl.BlockSpec((ROWS, FINE_INPUT_DIM), lambda i: (i, 0)),
            pl.BlockSpec((SCALAR_PROJ_DIM, OUTPUT_DIM), lambda i: (0, 0)),
            pl.BlockSpec((FINE_INPUT_DIM, OUTPUT_DIM), lambda i: (0, 0)),
            pl.BlockSpec((1, OUTPUT_DIM), lambda i: (0, 0)),
        ],
        out_specs=pl.BlockSpec((ROWS, OUTPUT_DIM), lambda i: (i, 0)),
        out_shape=jax.ShapeDtypeStruct((V_FINE, OUTPUT_DIM), jnp.float32),
        scratch_shapes=[pltpu.VMEM((V_COARSE, OUTPUT_DIM), jnp.bfloat16)],
        compiler_params=pltpu.CompilerParams(
            dimension_semantics=("arbitrary",)),
    )(x_coarse, W_sym, b_sym2, interp_matrix, x_fine_input,
      w_fuse1, w_fuse2, b_fuse2)
